# baseline (speedup 1.0000x reference)
import jax
import jax.numpy as jnp
from jax import lax
from jax.experimental import pallas as pl
from jax.experimental.pallas import tpu as pltpu
from jax.experimental.pallas import tpu_sc as plsc

_L = 16
_NC = 2
_NS = 16
_NW = _NC * _NS
_B = 16384
_D = 64
_PD = 2 * _D
_BPW = _B // _NW
_CHUNK = 128
_NCH = _BPW // _CHUNK


def _sc_body(uidx_hbm, iidx_hbm, utab_hbm, itab_hbm, out_hbm,
             uidx_v, iidx_v, updx_v, ipdx_v, ubuf, ibuf, out_v, sems):
    wid = lax.axis_index("s") * _NC + lax.axis_index("c")
    base = wid * _BPW

    pltpu.sync_copy(uidx_hbm.at[pl.ds(base, _BPW)], uidx_v)
    pltpu.sync_copy(iidx_hbm.at[pl.ds(base, _BPW)], iidx_v)

    def pack_body(g, _):
        sl = pl.ds(g * _L, _L)
        updx_v[sl] = lax.shift_right_logical(uidx_v[sl], 1)
        ipdx_v[sl] = lax.shift_right_logical(iidx_v[sl], 1)
        return _

    lax.fori_loop(0, _BPW // _L, pack_body, 0)

    def fire(c):
        s = sems.at[c % 2]
        return (
            pltpu.async_copy(utab_hbm.at[updx_v.at[pl.ds(c * _CHUNK, _CHUNK)]],
                             ubuf.at[c % 2], s),
            pltpu.async_copy(itab_hbm.at[ipdx_v.at[pl.ds(c * _CHUNK, _CHUNK)]],
                             ibuf.at[c % 2], s),
        )

    lane = lax.iota(jnp.int32, _L)
    rots = [jnp.bitwise_and(lane + (1 << t), _L - 1) for t in range(4)]

    def compute_chunk(c):
        ub = ubuf.at[c % 2]
        ib = ibuf.at[c % 2]

        def group_body(g, _):
            r0 = g * _L
            gsl = pl.ds(c * _CHUNK + r0, _L)
            huv = (uidx_v[gsl] & 1) * _D
            hiv = (iidx_v[gsl] & 1) * _D
            accv = jnp.zeros((_L,), jnp.float32)
            for j in range(_L):
                r = r0 + j
                hu = huv[j]
                hi = hiv[j]
                p = (ub[r, pl.ds(hu, _L)] * ib[r, pl.ds(hi, _L)])
                for k in range(1, _D // _L):
                    p = p + (ub[r, pl.ds(hu + k * _L, _L)]
                             * ib[r, pl.ds(hi + k * _L, _L)])
                for t in range(4):
                    p = p + jnp.take(p, rots[t], axis=0)
                accv = jnp.where(lane == j, p, accv)
            out_v[pl.ds(c * _CHUNK + r0, _L)] = accv
            return _

        lax.fori_loop(0, _CHUNK // _L, group_body, 0)

    pending = fire(0)
    for c in range(_NCH):
        nxt = fire(c + 1) if c + 1 < _NCH else ()
        for cp in pending:
            cp.wait()
        compute_chunk(c)
        pending = nxt

    pltpu.sync_copy(out_v, out_hbm.at[pl.ds(base, _BPW)])


@jax.jit
def _run(user_indices, item_indices, utab2, itab2):
    mesh = plsc.VectorSubcoreMesh(core_axis_name="c", subcore_axis_name="s")
    f = pl.kernel(
        _sc_body,
        mesh=mesh,
        out_type=jax.ShapeDtypeStruct((_B,), jnp.float32),
        scratch_types=[
            pltpu.VMEM((_BPW,), jnp.int32),
            pltpu.VMEM((_BPW,), jnp.int32),
            pltpu.VMEM((_BPW,), jnp.int32),
            pltpu.VMEM((_BPW,), jnp.int32),
            pltpu.VMEM((2, _CHUNK, _PD), jnp.float32),
            pltpu.VMEM((2, _CHUNK, _PD), jnp.float32),
            pltpu.VMEM((_BPW,), jnp.float32),
            pltpu.SemaphoreType.DMA((2,)),
        ],
    )
    return f(user_indices, item_indices, utab2, itab2)


def kernel(user_indices, item_indices, user_table, item_table):
    utab2 = user_table.reshape(-1, _PD)
    itab2 = item_table.reshape(-1, _PD)
    return _run(user_indices.astype(jnp.int32), item_indices.astype(jnp.int32),
                utab2, itab2)
